# baseline (speedup 1.0000x reference)
import jax
import jax.numpy as jnp
from jax import lax
from jax.experimental import pallas as pl
from jax.experimental.pallas import tpu as pltpu
from jax.experimental.pallas import tpu_sc as plsc

E = 8
TOPK = 2
N = 2048
NP = N * TOPK
H = 768
FF = 3072
BLK = 128
NBLK = NP // BLK + E
NPAD = NBLK * BLK
BE_PAD = 48
NC = 2
NS = 16
NW = NC * NS
CHUNK = NP // NW
LANES = 16
WSW = 128

_MESH = dict(core_axis_name="c", subcore_axis_name="s")
_SC_PARAMS = pltpu.CompilerParams(needs_layout_passes=False)


def _wid():
    return lax.axis_index("s") * NC + lax.axis_index("c")


def _bc(x):
    return jnp.broadcast_to(x, (LANES,))


def _cv(val):
    return jnp.full((LANES,), val, jnp.int32)


def _zv():
    return jnp.zeros((LANES,), jnp.int32)


def _iota():
    return lax.iota(jnp.int32, LANES)


_GDN = lax.GatherDimensionNumbers(
    offset_dims=(), collapsed_slice_dims=(0,), start_index_map=(0,))


def _splat(vec, e):
    idx = _cv(e)
    return lax.gather(vec, idx[:, None], _GDN, (1,),
                      mode=lax.GatherScatterMode.PROMISE_IN_BOUNDS)


def _hist_body(ids_hbm, tbl_hbm, ids_v, row_v):
    wid = _wid()
    pltpu.sync_copy(ids_hbm.at[pl.ds(wid * CHUNK, CHUNK)], ids_v)
    iota = _iota()
    cvec = _zv()
    for e in range(E):
        te = _zv()
        for r in range(CHUNK // LANES):
            v = ids_v[pl.ds(r * LANES, LANES)]
            te = te + jnp.where(v == _cv(e), _cv(1), _zv())
        cs = plsc.cumsum(te)
        cvec = jnp.where(iota == _cv(e), _splat(cs, LANES - 1), cvec)
    row_v[...] = cvec
    pltpu.sync_copy(row_v, tbl_hbm.at[wid])


def _route_body(ids_hbm, x_hbm, tbl_hbm, wtab_hbm, ew_hbm,
                xs_hbm, slots_hbm, be_hbm, ws_hbm,
                ids_v, tbl_v, wid_v, slots_v, tok_v, rows_v, be_v,
                ew_v, roww_v, sem):
    wid = _wid()
    iota = _iota()
    pltpu.sync_copy(tbl_hbm, tbl_v)
    pltpu.sync_copy(ids_hbm.at[pl.ds(wid * CHUNK, CHUNK)], ids_v)
    pltpu.sync_copy(ew_hbm.at[pl.ds(wid * CHUNK, CHUNK)], ew_v)
    pltpu.sync_copy(wtab_hbm.at[pl.ds(wid * LANES, LANES)], wid_v)
    widv = wid_v[...]

    pc = _zv()
    tot = _zv()
    for c in range(NW):
        row = tbl_v[c]
        tot = tot + row
        pc = pc + jnp.where(_cv(c) < widv, row, _zv())

    padded = ((tot + _cv(BLK - 1)) // _cv(BLK)) * _cv(BLK)
    csum = plsc.cumsum(padded)
    offs = csum - padded
    cb = offs + pc
    bend = csum // _cv(BLK)
    cbv = [_splat(cb, e) for e in range(E)]

    runs = [_zv()] * E
    for r in range(CHUNK // LANES):
        v = ids_v[pl.ds(r * LANES, LANES)]
        slot_r = _zv()
        for e in range(E):
            m = v == _cv(e)
            mi = jnp.where(m, _cv(1), _zv())
            cs = plsc.cumsum(mi)
            rank = runs[e] + (cs - mi)
            slot_r = jnp.where(m, cbv[e] + rank, slot_r)
            runs[e] = runs[e] + _splat(cs, LANES - 1)
        slots_v[pl.ds(r * LANES, LANES)] = slot_r
        tok_v[pl.ds(r * LANES, LANES)] = (
            _cv(r * LANES) + widv * _cv(CHUNK) + iota) % _cv(N)

    pltpu.sync_copy(slots_v, slots_hbm.at[pl.ds(wid * CHUNK, CHUNK)])
    for r in range(CHUNK // LANES):
        wv = ew_v[pl.ds(r * LANES, LANES)]
        plsc.store_scatter(roww_v, [iota + _cv(r * LANES), _zv()], wv)
    pltpu.async_copy(roww_v, ws_hbm.at[slots_v], sem).wait()
    pltpu.async_copy(x_hbm.at[tok_v], rows_v, sem).wait()
    pltpu.async_copy(rows_v, xs_hbm.at[slots_v], sem).wait()

    @pl.when(wid == 0)
    def _():
        for g in range(BE_PAD // LANES):
            b = _cv(g * LANES) + iota
            acc = _zv()
            for e in range(E):
                acc = acc + jnp.where(b >= _splat(bend, e), _cv(1), _zv())
            be_v[pl.ds(g * LANES, LANES)] = jnp.minimum(acc, _cv(E - 1))
        pltpu.sync_copy(be_v, be_hbm)


TPW = N // NW


def _final_body(y_hbm, slots_hbm, out_hbm,
                slots0_v, slots1_v, rows0_v, rows1_v, sem0, sem1):
    wid = _wid()
    pltpu.sync_copy(slots_hbm.at[pl.ds(wid * TPW, TPW)], slots0_v)
    pltpu.sync_copy(slots_hbm.at[pl.ds(N + wid * TPW, TPW)], slots1_v)
    c0 = pltpu.async_copy(y_hbm.at[slots0_v], rows0_v, sem0)
    c1 = pltpu.async_copy(y_hbm.at[slots1_v], rows1_v, sem1)
    c0.wait()
    c1.wait()

    def _acc(r, carry):
        for c in range(H // LANES):
            plsc.addupdate(rows0_v.at[r, pl.ds(c * LANES, LANES)],
                           rows1_v[r, pl.ds(c * LANES, LANES)])
        return carry

    lax.fori_loop(0, TPW, _acc, jnp.int32(0))
    pltpu.sync_copy(rows0_v, out_hbm.at[pl.ds(wid * TPW, TPW)])


def _w_copy(w1_hbm, w2_hbm, w1b, w2b, sem1, sem2, j):
    return (pltpu.make_async_copy(w1_hbm.at[j], w1b.at[j % 2], sem1),
            pltpu.make_async_copy(w2_hbm.at[j], w2b.at[j % 2], sem2))


def _mlp_body(be_ref, x_ref, ws_ref, w1_hbm, w2_hbm, y_ref, w1b, w2b, st_ref,
              sem1, sem2):
    i = pl.program_id(0)

    @pl.when(i == 0)
    def _():
        st_ref[0] = 0
        st_ref[1] = 0

    e = be_ref[i]
    prev = jnp.where(i == 0, jnp.int32(-1), be_ref[jnp.maximum(i - 1, 0)])

    @pl.when(e != prev)
    def _():
        issued = st_ref[0]
        waited = st_ref[1]
        target = jnp.minimum(e + 2, E)
        for j in range(E + 1):
            if j < E:
                @pl.when((j >= issued) & (j < target))
                def _(j=j):
                    c1, c2 = _w_copy(w1_hbm, w2_hbm, w1b, w2b, sem1, sem2, j)
                    c1.start()
                    c2.start()
            if j >= 1:
                @pl.when((j - 1 >= waited) & (j - 1 <= e))
                def _(j=j):
                    c1, c2 = _w_copy(w1_hbm, w2_hbm, w1b, w2b, sem1, sem2,
                                     j - 1)
                    c1.wait()
                    c2.wait()
        st_ref[0] = jnp.maximum(issued, target)
        st_ref[1] = jnp.maximum(waited, e + 1)

    e2 = lax.rem(e, 2)
    h = jnp.dot(x_ref[...], w1b[e2], preferred_element_type=jnp.float32)
    h = jax.nn.gelu(h)
    y = jnp.dot(h, w2b[e2], preferred_element_type=jnp.float32)
    y_ref[...] = y * ws_ref[:, 0:1]


def kernel(x, expert_weights, expert_indices, scores, w1, w2):
    del scores
    sl, bs, h = x.shape
    x_flat = x.reshape(N, H)
    ids = expert_indices.T.reshape(NP)
    ew_t = expert_weights.T.reshape(NP)

    mesh = plsc.VectorSubcoreMesh(**_MESH)

    hist = pl.kernel(
        _hist_body,
        out_type=jax.ShapeDtypeStruct((NW, LANES), jnp.int32),
        mesh=mesh,
        scratch_types=[
            pltpu.VMEM((CHUNK,), jnp.int32),
            pltpu.VMEM((LANES,), jnp.int32),
        ],
        compiler_params=_SC_PARAMS,
    )
    tbl = hist(ids)

    route = pl.kernel(
        _route_body,
        out_type=(
            jax.ShapeDtypeStruct((NPAD, H), jnp.float32),
            jax.ShapeDtypeStruct((NP,), jnp.int32),
            jax.ShapeDtypeStruct((BE_PAD,), jnp.int32),
            jax.ShapeDtypeStruct((NPAD, WSW), jnp.float32),
        ),
        mesh=mesh,
        scratch_types=[
            pltpu.VMEM((CHUNK,), jnp.int32),
            pltpu.VMEM((NW, LANES), jnp.int32),
            pltpu.VMEM((LANES,), jnp.int32),
            pltpu.VMEM((CHUNK,), jnp.int32),
            pltpu.VMEM((CHUNK,), jnp.int32),
            pltpu.VMEM((CHUNK, H), jnp.float32),
            pltpu.VMEM((BE_PAD,), jnp.int32),
            pltpu.VMEM((CHUNK,), jnp.float32),
            pltpu.VMEM((CHUNK, WSW), jnp.float32),
            pltpu.SemaphoreType.DMA,
        ],
        compiler_params=_SC_PARAMS,
    )
    wtab = jnp.repeat(jnp.arange(NW, dtype=jnp.int32), LANES)
    xs, slots, be, ws = route(ids, x_flat, tbl, wtab, ew_t)

    y_sorted = pl.pallas_call(
        _mlp_body,
        grid_spec=pltpu.PrefetchScalarGridSpec(
            num_scalar_prefetch=1,
            grid=(NBLK,),
            in_specs=[
                pl.BlockSpec((BLK, H), lambda i, be_r: (i, 0)),
                pl.BlockSpec((BLK, WSW), lambda i, be_r: (i, 0)),
                pl.BlockSpec(memory_space=pl.ANY),
                pl.BlockSpec(memory_space=pl.ANY),
            ],
            out_specs=pl.BlockSpec((BLK, H), lambda i, be_r: (i, 0)),
            scratch_shapes=[
                pltpu.VMEM((2, H, FF), jnp.float32),
                pltpu.VMEM((2, FF, H), jnp.float32),
                pltpu.SMEM((2,), jnp.int32),
                pltpu.SemaphoreType.DMA,
                pltpu.SemaphoreType.DMA,
            ],
        ),
        out_shape=jax.ShapeDtypeStruct((NPAD, H), jnp.float32),
    )(be, xs, ws, w1, w2)

    final = pl.kernel(
        _final_body,
        out_type=jax.ShapeDtypeStruct((N, H), jnp.float32),
        mesh=mesh,
        scratch_types=[
            pltpu.VMEM((TPW,), jnp.int32),
            pltpu.VMEM((TPW,), jnp.int32),
            pltpu.VMEM((TPW, H), jnp.float32),
            pltpu.VMEM((TPW, H), jnp.float32),
            pltpu.SemaphoreType.DMA,
            pltpu.SemaphoreType.DMA,
        ],
        compiler_params=_SC_PARAMS,
    )
    out = final(y_sorted, slots)

    return out.reshape(sl, bs, h)


# speedup vs baseline: 1.1463x; 1.1463x over previous
import jax
import jax.numpy as jnp
from jax import lax
from jax.experimental import pallas as pl
from jax.experimental.pallas import tpu as pltpu
from jax.experimental.pallas import tpu_sc as plsc

E = 8
TOPK = 2
N = 2048
NP = N * TOPK
H = 768
FF = 3072
BLK = 256
NBLK = NP // BLK + E
NPAD = NBLK * BLK
BE_PAD = 32
NC = 2
NS = 16
NW = NC * NS
CHUNK = NP // NW
LANES = 16
WSW = 128

_MESH = dict(core_axis_name="c", subcore_axis_name="s")
_SC_PARAMS = pltpu.CompilerParams(needs_layout_passes=False)


def _wid():
    return lax.axis_index("s") * NC + lax.axis_index("c")


def _bc(x):
    return jnp.broadcast_to(x, (LANES,))


def _cv(val):
    return jnp.full((LANES,), val, jnp.int32)


def _zv():
    return jnp.zeros((LANES,), jnp.int32)


def _iota():
    return lax.iota(jnp.int32, LANES)


_GDN = lax.GatherDimensionNumbers(
    offset_dims=(), collapsed_slice_dims=(0,), start_index_map=(0,))


def _splat(vec, e):
    idx = _cv(e)
    return lax.gather(vec, idx[:, None], _GDN, (1,),
                      mode=lax.GatherScatterMode.PROMISE_IN_BOUNDS)


def _hist_body(ids_ref, tbl_ref, be_ref):
    ids = ids_ref[...]
    li = lax.broadcasted_iota(jnp.int32, (NW, LANES), 1)
    acc = jnp.zeros((NW, LANES), jnp.int32)
    for e in range(E):
        cnt = jnp.sum((ids == e).astype(jnp.int32), axis=1, keepdims=True)
        acc = jnp.where(li == e, cnt, acc)
    tbl_ref[...] = acc
    b = lax.broadcasted_iota(jnp.int32, (BE_PAD,), 0)
    accb = jnp.zeros((BE_PAD,), jnp.int32)
    run = jnp.int32(0)
    for e in range(E):
        tot_e = jnp.sum(acc[:, e])
        run = run + (tot_e + BLK - 1) // BLK
        accb = accb + (b >= run).astype(jnp.int32)
    be_ref[...] = jnp.minimum(accb, E - 1)


def _route_body(ids_hbm, x_hbm, tbl_hbm, wtab_hbm, ew_hbm,
                xs_hbm, slots_hbm, ws_hbm,
                ids_v, tbl_v, wid_v, slots_v, tok_v, rows_v,
                ew_v, roww_v, sem):
    wid = _wid()
    iota = _iota()
    pltpu.sync_copy(tbl_hbm, tbl_v)
    pltpu.sync_copy(ids_hbm.at[pl.ds(wid * CHUNK, CHUNK)], ids_v)
    pltpu.sync_copy(ew_hbm.at[pl.ds(wid * CHUNK, CHUNK)], ew_v)
    pltpu.sync_copy(wtab_hbm.at[pl.ds(wid * LANES, LANES)], wid_v)
    widv = wid_v[...]

    pc = _zv()
    tot = _zv()
    for c in range(NW):
        row = tbl_v[c]
        tot = tot + row
        pc = pc + jnp.where(_cv(c) < widv, row, _zv())

    padded = ((tot + _cv(BLK - 1)) // _cv(BLK)) * _cv(BLK)
    csum = plsc.cumsum(padded)
    offs = csum - padded
    cb = offs + pc
    cbv = [_splat(cb, e) for e in range(E)]

    runs = [_zv()] * E
    for r in range(CHUNK // LANES):
        v = ids_v[pl.ds(r * LANES, LANES)]
        slot_r = _zv()
        for e in range(E):
            m = v == _cv(e)
            mi = jnp.where(m, _cv(1), _zv())
            cs = plsc.cumsum(mi)
            rank = runs[e] + (cs - mi)
            slot_r = jnp.where(m, cbv[e] + rank, slot_r)
            runs[e] = runs[e] + _splat(cs, LANES - 1)
        slots_v[pl.ds(r * LANES, LANES)] = slot_r
        tok_v[pl.ds(r * LANES, LANES)] = (
            _cv(r * LANES) + widv * _cv(CHUNK) + iota) % _cv(N)

    pltpu.sync_copy(slots_v, slots_hbm.at[pl.ds(wid * CHUNK, CHUNK)])
    for r in range(CHUNK // LANES):
        wv = ew_v[pl.ds(r * LANES, LANES)]
        plsc.store_scatter(roww_v, [iota + _cv(r * LANES), _zv()], wv)
    pltpu.async_copy(roww_v, ws_hbm.at[slots_v], sem).wait()
    pltpu.async_copy(x_hbm.at[tok_v], rows_v, sem).wait()
    pltpu.async_copy(rows_v, xs_hbm.at[slots_v], sem).wait()


TPW = N // NW


def _final_body(y_hbm, slots_hbm, out_hbm,
                slots0_v, slots1_v, rows0_v, rows1_v, sem0, sem1):
    wid = _wid()
    pltpu.sync_copy(slots_hbm.at[pl.ds(wid * TPW, TPW)], slots0_v)
    pltpu.sync_copy(slots_hbm.at[pl.ds(N + wid * TPW, TPW)], slots1_v)
    c0 = pltpu.async_copy(y_hbm.at[slots0_v], rows0_v, sem0)
    c1 = pltpu.async_copy(y_hbm.at[slots1_v], rows1_v, sem1)
    c0.wait()
    c1.wait()

    def _acc(r, carry):
        for c in range(H // LANES):
            plsc.addupdate(rows0_v.at[r, pl.ds(c * LANES, LANES)],
                           rows1_v[r, pl.ds(c * LANES, LANES)])
        return carry

    lax.fori_loop(0, TPW, _acc, jnp.int32(0))
    pltpu.sync_copy(rows0_v, out_hbm.at[pl.ds(wid * TPW, TPW)])


def _w_copy(w1_hbm, w2_hbm, w1b, w2b, sem1, sem2, j):
    return (pltpu.make_async_copy(w1_hbm.at[j], w1b.at[j % 2], sem1),
            pltpu.make_async_copy(w2_hbm.at[j], w2b.at[j % 2], sem2))


def _mlp_body(be_ref, x_ref, ws_ref, w1_hbm, w2_hbm, y_ref, w1b, w2b, st_ref,
              sem1, sem2):
    i = pl.program_id(0)

    @pl.when(i == 0)
    def _():
        st_ref[0] = 0
        st_ref[1] = 0

    e = be_ref[i]
    prev = jnp.where(i == 0, jnp.int32(-1), be_ref[jnp.maximum(i - 1, 0)])

    @pl.when(e != prev)
    def _():
        issued = st_ref[0]
        waited = st_ref[1]
        target = jnp.minimum(e + 2, E)
        for j in range(E + 1):
            if j < E:
                @pl.when((j >= issued) & (j < target))
                def _(j=j):
                    c1, c2 = _w_copy(w1_hbm, w2_hbm, w1b, w2b, sem1, sem2, j)
                    c1.start()
                    c2.start()
            if j >= 1:
                @pl.when((j - 1 >= waited) & (j - 1 <= e))
                def _(j=j):
                    c1, c2 = _w_copy(w1_hbm, w2_hbm, w1b, w2b, sem1, sem2,
                                     j - 1)
                    c1.wait()
                    c2.wait()
        st_ref[0] = jnp.maximum(issued, target)
        st_ref[1] = jnp.maximum(waited, e + 1)

    e2 = lax.rem(e, 2)
    h = jnp.dot(x_ref[...], w1b[e2], preferred_element_type=jnp.float32)
    h = jax.nn.gelu(h)
    y = jnp.dot(h, w2b[e2], preferred_element_type=jnp.float32)
    y_ref[...] = y * ws_ref[:, 0:1]


def kernel(x, expert_weights, expert_indices, scores, w1, w2):
    del scores
    sl, bs, h = x.shape
    x_flat = x.reshape(N, H)
    ids = expert_indices.T.reshape(NP)
    ew_t = expert_weights.T.reshape(NP)

    mesh = plsc.VectorSubcoreMesh(**_MESH)

    tbl, be = pl.pallas_call(
        _hist_body,
        out_shape=(
            jax.ShapeDtypeStruct((NW, LANES), jnp.int32),
            jax.ShapeDtypeStruct((BE_PAD,), jnp.int32),
        ),
    )(ids.reshape(NW, CHUNK))

    route = pl.kernel(
        _route_body,
        out_type=(
            jax.ShapeDtypeStruct((NPAD, H), jnp.float32),
            jax.ShapeDtypeStruct((NP,), jnp.int32),
            jax.ShapeDtypeStruct((NPAD, WSW), jnp.float32),
        ),
        mesh=mesh,
        scratch_types=[
            pltpu.VMEM((CHUNK,), jnp.int32),
            pltpu.VMEM((NW, LANES), jnp.int32),
            pltpu.VMEM((LANES,), jnp.int32),
            pltpu.VMEM((CHUNK,), jnp.int32),
            pltpu.VMEM((CHUNK,), jnp.int32),
            pltpu.VMEM((CHUNK, H), jnp.float32),
            pltpu.VMEM((CHUNK,), jnp.float32),
            pltpu.VMEM((CHUNK, WSW), jnp.float32),
            pltpu.SemaphoreType.DMA,
        ],
        compiler_params=_SC_PARAMS,
    )
    wtab = jnp.repeat(jnp.arange(NW, dtype=jnp.int32), LANES)
    xs, slots, ws = route(ids, x_flat, tbl, wtab, ew_t)

    y_sorted = pl.pallas_call(
        _mlp_body,
        grid_spec=pltpu.PrefetchScalarGridSpec(
            num_scalar_prefetch=1,
            grid=(NBLK,),
            in_specs=[
                pl.BlockSpec((BLK, H), lambda i, be_r: (i, 0)),
                pl.BlockSpec((BLK, WSW), lambda i, be_r: (i, 0)),
                pl.BlockSpec(memory_space=pl.ANY),
                pl.BlockSpec(memory_space=pl.ANY),
            ],
            out_specs=pl.BlockSpec((BLK, H), lambda i, be_r: (i, 0)),
            scratch_shapes=[
                pltpu.VMEM((2, H, FF), jnp.float32),
                pltpu.VMEM((2, FF, H), jnp.float32),
                pltpu.SMEM((2,), jnp.int32),
                pltpu.SemaphoreType.DMA,
                pltpu.SemaphoreType.DMA,
            ],
        ),
        out_shape=jax.ShapeDtypeStruct((NPAD, H), jnp.float32),
    )(be, xs, ws, w1, w2)

    final = pl.kernel(
        _final_body,
        out_type=jax.ShapeDtypeStruct((N, H), jnp.float32),
        mesh=mesh,
        scratch_types=[
            pltpu.VMEM((TPW,), jnp.int32),
            pltpu.VMEM((TPW,), jnp.int32),
            pltpu.VMEM((TPW, H), jnp.float32),
            pltpu.VMEM((TPW, H), jnp.float32),
            pltpu.SemaphoreType.DMA,
            pltpu.SemaphoreType.DMA,
        ],
        compiler_params=_SC_PARAMS,
    )
    out = final(y_sorted, slots)

    return out.reshape(sl, bs, h)


# speedup vs baseline: 1.1563x; 1.0087x over previous
import jax
import jax.numpy as jnp
from jax import lax
from jax.experimental import pallas as pl
from jax.experimental.pallas import tpu as pltpu
from jax.experimental.pallas import tpu_sc as plsc

E = 8
TOPK = 2
N = 2048
NP = N * TOPK
H = 768
FF = 3072
BLK = 256
NBLK = NP // BLK + E
NPAD = NBLK * BLK
BE_PAD = 32
NC = 2
NS = 16
NW = NC * NS
CHUNK = NP // NW
LANES = 16
WSW = 128

_MESH = dict(core_axis_name="c", subcore_axis_name="s")
_SC_PARAMS = pltpu.CompilerParams(needs_layout_passes=False)


def _wid():
    return lax.axis_index("s") * NC + lax.axis_index("c")


def _bc(x):
    return jnp.broadcast_to(x, (LANES,))


def _cv(val):
    return jnp.full((LANES,), val, jnp.int32)


def _zv():
    return jnp.zeros((LANES,), jnp.int32)


def _iota():
    return lax.iota(jnp.int32, LANES)


_GDN = lax.GatherDimensionNumbers(
    offset_dims=(), collapsed_slice_dims=(0,), start_index_map=(0,))


def _splat(vec, e):
    idx = _cv(e)
    return lax.gather(vec, idx[:, None], _GDN, (1,),
                      mode=lax.GatherScatterMode.PROMISE_IN_BOUNDS)


def _hist_body(ids_ref, tbl_ref, be_ref):
    ids = ids_ref[...]
    li = lax.broadcasted_iota(jnp.int32, (NW, LANES), 1)
    acc = jnp.zeros((NW, LANES), jnp.int32)
    for e in range(E):
        cnt = jnp.sum((ids == e).astype(jnp.int32), axis=1, keepdims=True)
        acc = jnp.where(li == e, cnt, acc)
    tbl_ref[...] = acc
    b = lax.broadcasted_iota(jnp.int32, (BE_PAD,), 0)
    accb = jnp.zeros((BE_PAD,), jnp.int32)
    run = jnp.int32(0)
    for e in range(E):
        tot_e = jnp.sum(acc[:, e])
        run = run + (tot_e + BLK - 1) // BLK
        accb = accb + (b >= run).astype(jnp.int32)
    be_ref[...] = jnp.minimum(accb, E - 1)


def _route_body(ids_hbm, x_hbm, tbl_hbm, wtab_hbm, ew_hbm,
                xs_hbm, slots_hbm, ws_hbm,
                ids_v, tbl_v, wid_v, slots_v, tok_v, rows_v,
                ew_v, roww_v, sem, sem2):
    wid = _wid()
    iota = _iota()
    pltpu.sync_copy(tbl_hbm, tbl_v)
    pltpu.sync_copy(ids_hbm.at[pl.ds(wid * CHUNK, CHUNK)], ids_v)
    pltpu.sync_copy(ew_hbm.at[pl.ds(wid * CHUNK, CHUNK)], ew_v)
    pltpu.sync_copy(wtab_hbm.at[pl.ds(wid * LANES, LANES)], wid_v)
    widv = wid_v[...]

    pc = _zv()
    tot = _zv()
    for c in range(NW):
        row = tbl_v[c]
        tot = tot + row
        pc = pc + jnp.where(_cv(c) < widv, row, _zv())

    padded = ((tot + _cv(BLK - 1)) // _cv(BLK)) * _cv(BLK)
    csum = plsc.cumsum(padded)
    offs = csum - padded
    cb = offs + pc
    cbv = [_splat(cb, e) for e in range(E)]

    runs = [_zv()] * E
    for r in range(CHUNK // LANES):
        v = ids_v[pl.ds(r * LANES, LANES)]
        slot_r = _zv()
        for e in range(E):
            m = v == _cv(e)
            mi = jnp.where(m, _cv(1), _zv())
            cs = plsc.cumsum(mi)
            rank = runs[e] + (cs - mi)
            slot_r = jnp.where(m, cbv[e] + rank, slot_r)
            runs[e] = runs[e] + _splat(cs, LANES - 1)
        slots_v[pl.ds(r * LANES, LANES)] = slot_r
        tok_v[pl.ds(r * LANES, LANES)] = (
            _cv(r * LANES) + widv * _cv(CHUNK) + iota) % _cv(N)

    pltpu.sync_copy(slots_v, slots_hbm.at[pl.ds(wid * CHUNK, CHUNK)])
    for r in range(CHUNK // LANES):
        wv = ew_v[pl.ds(r * LANES, LANES)]
        plsc.store_scatter(roww_v, [iota + _cv(r * LANES), _zv()], wv)
    c_ws = pltpu.async_copy(roww_v, ws_hbm.at[slots_v], sem2)
    pltpu.async_copy(x_hbm.at[tok_v], rows_v, sem).wait()
    pltpu.async_copy(rows_v, xs_hbm.at[slots_v], sem).wait()
    c_ws.wait()


TPW = N // NW


def _final_body(y_hbm, slots_hbm, out_hbm,
                slots0_v, slots1_v, rows0_v, rows1_v, sem0, sem1):
    wid = _wid()
    pltpu.sync_copy(slots_hbm.at[pl.ds(wid * TPW, TPW)], slots0_v)
    pltpu.sync_copy(slots_hbm.at[pl.ds(N + wid * TPW, TPW)], slots1_v)
    c0 = pltpu.async_copy(y_hbm.at[slots0_v], rows0_v, sem0)
    c1 = pltpu.async_copy(y_hbm.at[slots1_v], rows1_v, sem1)
    c0.wait()
    c1.wait()

    def _acc(r, carry):
        for c in range(H // LANES):
            plsc.addupdate(rows0_v.at[r, pl.ds(c * LANES, LANES)],
                           rows1_v[r, pl.ds(c * LANES, LANES)])
        return carry

    lax.fori_loop(0, TPW, _acc, jnp.int32(0))
    pltpu.sync_copy(rows0_v, out_hbm.at[pl.ds(wid * TPW, TPW)])


def _w_copy(w1_hbm, w2_hbm, w1b, w2b, sem1, sem2, j):
    return (pltpu.make_async_copy(w1_hbm.at[j], w1b.at[j % 2], sem1),
            pltpu.make_async_copy(w2_hbm.at[j], w2b.at[j % 2], sem2))


def _mlp_body(be_ref, x_ref, ws_ref, w1_hbm, w2_hbm, y_ref, w1b, w2b, st_ref,
              sem1, sem2):
    i = pl.program_id(0)

    @pl.when(i == 0)
    def _():
        st_ref[0] = 0
        st_ref[1] = 0

    e = be_ref[i]
    prev = jnp.where(i == 0, jnp.int32(-1), be_ref[jnp.maximum(i - 1, 0)])

    @pl.when(e != prev)
    def _():
        issued = st_ref[0]
        waited = st_ref[1]
        target = jnp.minimum(e + 2, E)
        for j in range(E + 1):
            if j < E:
                @pl.when((j >= issued) & (j < target))
                def _(j=j):
                    c1, c2 = _w_copy(w1_hbm, w2_hbm, w1b, w2b, sem1, sem2, j)
                    c1.start()
                    c2.start()
            if j >= 1:
                @pl.when((j - 1 >= waited) & (j - 1 <= e))
                def _(j=j):
                    c1, c2 = _w_copy(w1_hbm, w2_hbm, w1b, w2b, sem1, sem2,
                                     j - 1)
                    c1.wait()
                    c2.wait()
        st_ref[0] = jnp.maximum(issued, target)
        st_ref[1] = jnp.maximum(waited, e + 1)

    e2 = lax.rem(e, 2)
    h = jnp.dot(x_ref[...], w1b[e2], preferred_element_type=jnp.float32)
    h = jax.nn.gelu(h)
    y = jnp.dot(h, w2b[e2], preferred_element_type=jnp.float32)
    y_ref[...] = y * ws_ref[:, 0:1]


def kernel(x, expert_weights, expert_indices, scores, w1, w2):
    del scores
    sl, bs, h = x.shape
    x_flat = x.reshape(N, H)
    ids = expert_indices.T.reshape(NP)
    ew_t = expert_weights.T.reshape(NP)

    mesh = plsc.VectorSubcoreMesh(**_MESH)

    tbl, be = pl.pallas_call(
        _hist_body,
        out_shape=(
            jax.ShapeDtypeStruct((NW, LANES), jnp.int32),
            jax.ShapeDtypeStruct((BE_PAD,), jnp.int32),
        ),
    )(ids.reshape(NW, CHUNK))

    route = pl.kernel(
        _route_body,
        out_type=(
            jax.ShapeDtypeStruct((NPAD, H), jnp.float32),
            jax.ShapeDtypeStruct((NP,), jnp.int32),
            jax.ShapeDtypeStruct((NPAD, WSW), jnp.float32),
        ),
        mesh=mesh,
        scratch_types=[
            pltpu.VMEM((CHUNK,), jnp.int32),
            pltpu.VMEM((NW, LANES), jnp.int32),
            pltpu.VMEM((LANES,), jnp.int32),
            pltpu.VMEM((CHUNK,), jnp.int32),
            pltpu.VMEM((CHUNK,), jnp.int32),
            pltpu.VMEM((CHUNK, H), jnp.float32),
            pltpu.VMEM((CHUNK,), jnp.float32),
            pltpu.VMEM((CHUNK, WSW), jnp.float32),
            pltpu.SemaphoreType.DMA,
            pltpu.SemaphoreType.DMA,
        ],
        compiler_params=_SC_PARAMS,
    )
    wtab = jnp.repeat(jnp.arange(NW, dtype=jnp.int32), LANES)
    xs, slots, ws = route(ids, x_flat, tbl, wtab, ew_t)

    y_sorted = pl.pallas_call(
        _mlp_body,
        grid_spec=pltpu.PrefetchScalarGridSpec(
            num_scalar_prefetch=1,
            grid=(NBLK,),
            in_specs=[
                pl.BlockSpec((BLK, H), lambda i, be_r: (i, 0)),
                pl.BlockSpec((BLK, WSW), lambda i, be_r: (i, 0)),
                pl.BlockSpec(memory_space=pl.ANY),
                pl.BlockSpec(memory_space=pl.ANY),
            ],
            out_specs=pl.BlockSpec((BLK, H), lambda i, be_r: (i, 0)),
            scratch_shapes=[
                pltpu.VMEM((2, H, FF), jnp.float32),
                pltpu.VMEM((2, FF, H), jnp.float32),
                pltpu.SMEM((2,), jnp.int32),
                pltpu.SemaphoreType.DMA,
                pltpu.SemaphoreType.DMA,
            ],
        ),
        out_shape=jax.ShapeDtypeStruct((NPAD, H), jnp.float32),
    )(be, xs, ws, w1, w2)

    final = pl.kernel(
        _final_body,
        out_type=jax.ShapeDtypeStruct((N, H), jnp.float32),
        mesh=mesh,
        scratch_types=[
            pltpu.VMEM((TPW,), jnp.int32),
            pltpu.VMEM((TPW,), jnp.int32),
            pltpu.VMEM((TPW, H), jnp.float32),
            pltpu.VMEM((TPW, H), jnp.float32),
            pltpu.SemaphoreType.DMA,
            pltpu.SemaphoreType.DMA,
        ],
        compiler_params=_SC_PARAMS,
    )
    out = final(y_sorted, slots)

    return out.reshape(sl, bs, h)


# speedup vs baseline: 1.1571x; 1.0008x over previous
import jax
import jax.numpy as jnp
from jax import lax
from jax.experimental import pallas as pl
from jax.experimental.pallas import tpu as pltpu
from jax.experimental.pallas import tpu_sc as plsc

E = 8
TOPK = 2
N = 2048
NP = N * TOPK
H = 768
FF = 3072
BLK = 256
NBLK = NP // BLK + E
NPAD = NBLK * BLK
BE_PAD = 32
NC = 2
NS = 16
NW = NC * NS
CHUNK = NP // NW
LANES = 16
WSW = 128

_MESH = dict(core_axis_name="c", subcore_axis_name="s")
_SC_PARAMS = pltpu.CompilerParams(needs_layout_passes=False)


def _wid():
    return lax.axis_index("s") * NC + lax.axis_index("c")


def _bc(x):
    return jnp.broadcast_to(x, (LANES,))


def _cv(val):
    return jnp.full((LANES,), val, jnp.int32)


def _zv():
    return jnp.zeros((LANES,), jnp.int32)


def _iota():
    return lax.iota(jnp.int32, LANES)


_GDN = lax.GatherDimensionNumbers(
    offset_dims=(), collapsed_slice_dims=(0,), start_index_map=(0,))


def _splat(vec, e):
    idx = _cv(e)
    return lax.gather(vec, idx[:, None], _GDN, (1,),
                      mode=lax.GatherScatterMode.PROMISE_IN_BOUNDS)


def _hist_body(ids_ref, tbl_ref, be_ref):
    ids = ids_ref[...]
    li = lax.broadcasted_iota(jnp.int32, (NW, LANES), 1)
    acc = jnp.zeros((NW, LANES), jnp.int32)
    for e in range(E):
        cnt = jnp.sum((ids == e).astype(jnp.int32), axis=1, keepdims=True)
        acc = jnp.where(li == e, cnt, acc)
    tbl_ref[...] = acc
    b = lax.broadcasted_iota(jnp.int32, (BE_PAD,), 0)
    accb = jnp.zeros((BE_PAD,), jnp.int32)
    run = jnp.int32(0)
    for e in range(E):
        tot_e = jnp.sum(acc[:, e])
        run = run + (tot_e + BLK - 1) // BLK
        accb = accb + (b >= run).astype(jnp.int32)
    be_ref[...] = jnp.minimum(accb, E - 1)


def _route_body(ids_hbm, x_hbm, tbl_hbm, wtab_hbm, ew_hbm,
                xs_hbm, slots_hbm, ws_hbm,
                ids_v, tbl_v, wid_v, slots_v, tok_v, rows_v,
                ew_v, roww_v, sem, sem2):
    wid = _wid()
    iota = _iota()
    pltpu.sync_copy(tbl_hbm, tbl_v)
    pltpu.sync_copy(ids_hbm.at[pl.ds(wid * CHUNK, CHUNK)], ids_v)
    pltpu.sync_copy(ew_hbm.at[pl.ds(wid * CHUNK, CHUNK)], ew_v)
    pltpu.sync_copy(wtab_hbm.at[pl.ds(wid * LANES, LANES)], wid_v)
    widv = wid_v[...]

    pc = _zv()
    tot = _zv()
    for c in range(NW):
        row = tbl_v[c]
        tot = tot + row
        pc = pc + jnp.where(_cv(c) < widv, row, _zv())

    padded = ((tot + _cv(BLK - 1)) // _cv(BLK)) * _cv(BLK)
    csum = plsc.cumsum(padded)
    offs = csum - padded
    cb = offs + pc
    cbv = [_splat(cb, e) for e in range(E)]

    runs = [_zv()] * E
    for r in range(CHUNK // LANES):
        v = ids_v[pl.ds(r * LANES, LANES)]
        slot_r = _zv()
        for e in range(E):
            m = v == _cv(e)
            mi = jnp.where(m, _cv(1), _zv())
            cs = plsc.cumsum(mi)
            rank = runs[e] + (cs - mi)
            slot_r = jnp.where(m, cbv[e] + rank, slot_r)
            runs[e] = runs[e] + _splat(cs, LANES - 1)
        slots_v[pl.ds(r * LANES, LANES)] = slot_r
        tok_v[pl.ds(r * LANES, LANES)] = (
            _cv(r * LANES) + widv * _cv(CHUNK) + iota) % _cv(N)

    c_gx = pltpu.async_copy(x_hbm.at[tok_v], rows_v, sem)
    pltpu.sync_copy(slots_v, slots_hbm.at[pl.ds(wid * CHUNK, CHUNK)])
    for r in range(CHUNK // LANES):
        wv = ew_v[pl.ds(r * LANES, LANES)]
        plsc.store_scatter(roww_v, [iota + _cv(r * LANES), _zv()], wv)
    c_ws = pltpu.async_copy(roww_v, ws_hbm.at[slots_v], sem2)
    c_gx.wait()
    pltpu.async_copy(rows_v, xs_hbm.at[slots_v], sem).wait()
    c_ws.wait()


TPW = N // NW


def _final_body(y_hbm, slots_hbm, out_hbm,
                slots0_v, slots1_v, rows0_v, rows1_v, sem0, sem1):
    wid = _wid()
    pltpu.sync_copy(slots_hbm.at[pl.ds(wid * TPW, TPW)], slots0_v)
    pltpu.sync_copy(slots_hbm.at[pl.ds(N + wid * TPW, TPW)], slots1_v)
    c0 = pltpu.async_copy(y_hbm.at[slots0_v], rows0_v, sem0)
    c1 = pltpu.async_copy(y_hbm.at[slots1_v], rows1_v, sem1)
    c0.wait()
    c1.wait()

    def _acc(r, carry):
        for c in range(H // LANES):
            plsc.addupdate(rows0_v.at[r, pl.ds(c * LANES, LANES)],
                           rows1_v[r, pl.ds(c * LANES, LANES)])
        return carry

    lax.fori_loop(0, TPW, _acc, jnp.int32(0))
    pltpu.sync_copy(rows0_v, out_hbm.at[pl.ds(wid * TPW, TPW)])


def _w_copy(w1_hbm, w2_hbm, w1b, w2b, sem1, sem2, j):
    return (pltpu.make_async_copy(w1_hbm.at[j], w1b.at[j % 2], sem1),
            pltpu.make_async_copy(w2_hbm.at[j], w2b.at[j % 2], sem2))


def _mlp_body(be_ref, x_ref, ws_ref, w1_hbm, w2_hbm, y_ref, w1b, w2b, st_ref,
              sem1, sem2):
    i = pl.program_id(0)

    @pl.when(i == 0)
    def _():
        st_ref[0] = 0
        st_ref[1] = 0

    e = be_ref[i]
    prev = jnp.where(i == 0, jnp.int32(-1), be_ref[jnp.maximum(i - 1, 0)])

    @pl.when(e != prev)
    def _():
        issued = st_ref[0]
        waited = st_ref[1]
        target = jnp.minimum(e + 2, E)
        for j in range(E + 1):
            if j < E:
                @pl.when((j >= issued) & (j < target))
                def _(j=j):
                    c1, c2 = _w_copy(w1_hbm, w2_hbm, w1b, w2b, sem1, sem2, j)
                    c1.start()
                    c2.start()
            if j >= 1:
                @pl.when((j - 1 >= waited) & (j - 1 <= e))
                def _(j=j):
                    c1, c2 = _w_copy(w1_hbm, w2_hbm, w1b, w2b, sem1, sem2,
                                     j - 1)
                    c1.wait()
                    c2.wait()
        st_ref[0] = jnp.maximum(issued, target)
        st_ref[1] = jnp.maximum(waited, e + 1)

    e2 = lax.rem(e, 2)
    h = jnp.dot(x_ref[...], w1b[e2], preferred_element_type=jnp.float32)
    h = jax.nn.gelu(h)
    y = jnp.dot(h, w2b[e2], preferred_element_type=jnp.float32)
    y_ref[...] = y * ws_ref[:, 0:1]


def kernel(x, expert_weights, expert_indices, scores, w1, w2):
    del scores
    sl, bs, h = x.shape
    x_flat = x.reshape(N, H)
    ids = expert_indices.T.reshape(NP)
    ew_t = expert_weights.T.reshape(NP)

    mesh = plsc.VectorSubcoreMesh(**_MESH)

    tbl, be = pl.pallas_call(
        _hist_body,
        out_shape=(
            jax.ShapeDtypeStruct((NW, LANES), jnp.int32),
            jax.ShapeDtypeStruct((BE_PAD,), jnp.int32),
        ),
    )(ids.reshape(NW, CHUNK))

    route = pl.kernel(
        _route_body,
        out_type=(
            jax.ShapeDtypeStruct((NPAD, H), jnp.float32),
            jax.ShapeDtypeStruct((NP,), jnp.int32),
            jax.ShapeDtypeStruct((NPAD, WSW), jnp.float32),
        ),
        mesh=mesh,
        scratch_types=[
            pltpu.VMEM((CHUNK,), jnp.int32),
            pltpu.VMEM((NW, LANES), jnp.int32),
            pltpu.VMEM((LANES,), jnp.int32),
            pltpu.VMEM((CHUNK,), jnp.int32),
            pltpu.VMEM((CHUNK,), jnp.int32),
            pltpu.VMEM((CHUNK, H), jnp.float32),
            pltpu.VMEM((CHUNK,), jnp.float32),
            pltpu.VMEM((CHUNK, WSW), jnp.float32),
            pltpu.SemaphoreType.DMA,
            pltpu.SemaphoreType.DMA,
        ],
        compiler_params=_SC_PARAMS,
    )
    wtab = jnp.repeat(jnp.arange(NW, dtype=jnp.int32), LANES)
    xs, slots, ws = route(ids, x_flat, tbl, wtab, ew_t)

    y_sorted = pl.pallas_call(
        _mlp_body,
        grid_spec=pltpu.PrefetchScalarGridSpec(
            num_scalar_prefetch=1,
            grid=(NBLK,),
            in_specs=[
                pl.BlockSpec((BLK, H), lambda i, be_r: (i, 0)),
                pl.BlockSpec((BLK, WSW), lambda i, be_r: (i, 0)),
                pl.BlockSpec(memory_space=pl.ANY),
                pl.BlockSpec(memory_space=pl.ANY),
            ],
            out_specs=pl.BlockSpec((BLK, H), lambda i, be_r: (i, 0)),
            scratch_shapes=[
                pltpu.VMEM((2, H, FF), jnp.float32),
                pltpu.VMEM((2, FF, H), jnp.float32),
                pltpu.SMEM((2,), jnp.int32),
                pltpu.SemaphoreType.DMA,
                pltpu.SemaphoreType.DMA,
            ],
        ),
        out_shape=jax.ShapeDtypeStruct((NPAD, H), jnp.float32),
    )(be, xs, ws, w1, w2)

    final = pl.kernel(
        _final_body,
        out_type=jax.ShapeDtypeStruct((N, H), jnp.float32),
        mesh=mesh,
        scratch_types=[
            pltpu.VMEM((TPW,), jnp.int32),
            pltpu.VMEM((TPW,), jnp.int32),
            pltpu.VMEM((TPW, H), jnp.float32),
            pltpu.VMEM((TPW, H), jnp.float32),
            pltpu.SemaphoreType.DMA,
            pltpu.SemaphoreType.DMA,
        ],
        compiler_params=_SC_PARAMS,
    )
    out = final(y_sorted, slots)

    return out.reshape(sl, bs, h)
